# Initial kernel scaffold; baseline (speedup 1.0000x reference)
import jax
import jax.numpy as jnp
from jax.experimental import pallas as pl


def kernel(heter_feature_2d, W_enc, b_enc, W_q, b_q, W_lat, b_lat, W_deq, b_deq, W_res, b_res, W_side, b_side, codebooks):
    raise NotImplementedError("write your pallas kernel here")


# speedup vs baseline: 12.6563x; 12.6563x over previous
import functools

import jax
import jax.numpy as jnp
from jax.experimental import pallas as pl
from jax.experimental.pallas import tpu as pltpu

CHANNEL = 64
SEG = 4
K = 128
LEVELS = 3
D = CHANNEL // SEG
KT = SEG * K


def _mm(a, b, dims):
    return jax.lax.dot_general(
        a, b, (dims, ((), ())), preferred_element_type=jnp.float32)


def _body(x_ref, w_enc, b_enc, w_q, b_q, w_lat, b_lat, w_deq, b_deq,
          w_res, b_res, w_side, b_side, bd_ref, bg_ref, csq_ref,
          out_ref, loss_ref, *, blk):
    i = pl.program_id(0)
    x = x_ref[...]
    loss = jnp.float32(0.0)
    hards = []
    cur = None
    for l in range(LEVELS):
        if l == 0:
            z = _mm(x, w_enc[l], ((0,), (1,)))
        else:
            z = _mm(cur, w_enc[l], ((1,), (1,)))
        z = z + b_enc[l][None, :]
        q = _mm(z, w_q[l], ((1,), (1,))) + b_q[l][None, :]
        cross = _mm(q, bd_ref[l], ((1,), (0,)))
        oh_parts = []
        for s in range(SEG):
            qs = q[:, D * s:D * (s + 1)]
            qsq = jnp.sum(qs * qs, axis=1, keepdims=True)
            ds = (qsq - 2.0 * cross[:, K * s:K * (s + 1)]) \
                + csq_ref[l][None, K * s:K * (s + 1)]
            mn = jnp.min(ds, axis=1, keepdims=True)
            iota = jax.lax.broadcasted_iota(jnp.int32, (blk, K), 1)
            idx = jnp.min(jnp.where(ds == mn, iota, K), axis=1, keepdims=True)
            oh_parts.append((iota == idx).astype(jnp.float32))
        oh = jnp.concatenate(oh_parts, axis=1)
        hard = _mm(oh, bg_ref[l], ((1,), (0,)))
        df = q - hard
        loss = loss + jnp.sum(df * df)
        hards.append(hard)
        cur = _mm(z, w_lat[l], ((1,), (1,))) + b_lat[l][None, :]
    t = (_mm(hards[2], w_deq[2], ((1,), (1,))) + b_deq[2][None, :]
         + b_side[2][None, :])
    y = _mm(t, w_res[2], ((1,), (1,))) + b_res[2][None, :]
    for l in (1, 0):
        t = (_mm(hards[l], w_deq[l], ((1,), (1,))) + b_deq[l][None, :]
             + _mm(y, w_side[l], ((1,), (1,))) + b_side[l][None, :])
        y = _mm(t, w_res[l], ((1,), (1,))) + b_res[l][None, :]
    out_ref[...] = y

    @pl.when(i == 0)
    def _init():
        loss_ref[0, 0] = loss

    @pl.when(i != 0)
    def _acc():
        loss_ref[0, 0] += loss


def kernel(heter_feature_2d, W_enc, b_enc, W_q, b_q, W_lat, b_lat,
           W_deq, b_deq, W_res, b_res, W_side, b_side, codebooks):
    Bq, C, Hq, Wq_ = heter_feature_2d.shape
    n = Bq * Hq * Wq_
    x = heter_feature_2d.reshape(C, n)
    blk = 1024
    grid = n // blk

    eye = jnp.eye(SEG, dtype=jnp.float32)
    cbT = jnp.transpose(codebooks, (0, 1, 3, 2))
    bd = jnp.einsum('lsdk,st->ltdsk', cbT, eye).reshape(LEVELS, C, KT)
    bg = jnp.transpose(bd, (0, 2, 1))
    csq = jnp.sum(codebooks * codebooks, axis=-1).reshape(LEVELS, KT)

    full = lambda shape: pl.BlockSpec(shape, lambda i: (0,) * len(shape))
    out, loss = pl.pallas_call(
        functools.partial(_body, blk=blk),
        grid=(grid,),
        in_specs=[
            pl.BlockSpec((C, blk), lambda i: (0, i)),
            full((LEVELS, C, C)), full((LEVELS, C)),
            full((LEVELS, C, C)), full((LEVELS, C)),
            full((LEVELS, C, C)), full((LEVELS, C)),
            full((LEVELS, C, C)), full((LEVELS, C)),
            full((LEVELS, C, C)), full((LEVELS, C)),
            full((LEVELS, C, C)), full((LEVELS, C)),
            full((LEVELS, C, KT)), full((LEVELS, KT, C)),
            full((LEVELS, KT)),
        ],
        out_specs=[
            pl.BlockSpec((blk, C), lambda i: (i, 0)),
            pl.BlockSpec((1, 1), lambda i: (0, 0),
                         memory_space=pltpu.SMEM),
        ],
        out_shape=[
            jax.ShapeDtypeStruct((n, C), jnp.float32),
            jax.ShapeDtypeStruct((1, 1), jnp.float32),
        ],
    )(x, W_enc, b_enc, W_q, b_q, W_lat, b_lat, W_deq, b_deq,
      W_res, b_res, W_side, b_side, bd, bg, csq)

    restored = out.reshape(Bq, Hq, Wq_, C)
    codebook_loss = loss[0, 0] * jnp.float32(1.25) / jnp.float32(n * C)
    return (restored, codebook_loss)
